# baseline (speedup 1.0000x reference)
import functools

import jax
import jax.numpy as jnp
from jax import lax
from jax.experimental import pallas as pl
from jax.experimental.pallas import tpu as pltpu
from jax.experimental.pallas import tpu_sc as plsc

_NC = 2
_NS = 16
_NW = _NC * _NS

_B = 16384
_D = 64
_N = 1000000
_B_PER_W = _B // _NW
_NGRP = _B_PER_W // 16
_DEPTH = 8


def _gather_body(tt_hbm, idx_hbm, out_hbm, idx_v, q_v, r_v, stg_v,
                 b0, b1, b2, b3, b4, b5, b6, b7,
                 s0, s1, s2, s3, s4, s5, s6, s7):
    bufs = (b0, b1, b2, b3, b4, b5, b6, b7)
    sems = (s0, s1, s2, s3, s4, s5, s6, s7)
    wid = lax.axis_index("s") * _NC + lax.axis_index("c")
    base = wid * _B_PER_W
    pltpu.sync_copy(idx_hbm.at[pl.ds(base, _B_PER_W)], idx_v)

    def prep(jg, carry):
        v = idx_v[pl.ds(jg * 16, 16)]
        q_v[pl.ds(jg * 16, 16)] = lax.shift_right_logical(v, 7)
        r_v[pl.ds(jg * 16, 16)] = lax.bitwise_and(v, 127)
        return carry

    lax.fori_loop(0, _NGRP, prep, 0)

    def fetch(q, slot):
        pltpu.async_copy(
            tt_hbm.at[:, pl.ds(q * 128, 128)],
            bufs[slot],
            sems[slot],
        )

    qhead = q_v[pl.ds(0, 16)]
    for t in range(_DEPTH - 1):
        fetch(qhead[t], t % _DEPTH)

    lanes = lax.iota(jnp.int32, 16)
    z16 = jnp.zeros((16,), jnp.int32)

    def group(jg, carry):
        qv = q_v[pl.ds(jg * 16, 16)]
        rv = r_v[pl.ds(jg * 16, 16)]
        jg_next = lax.min(jg + 1, _NGRP - 1)
        qnext = q_v[pl.ds(jg_next * 16, 16)]
        for k in range(16):
            slot = k % _DEPTH
            if k + _DEPTH - 1 < 16:
                qa = qv[k + _DEPTH - 1]
            else:
                qa = qnext[k + _DEPTH - 1 - 16]
            fetch(qa, (k + _DEPTH - 1) % _DEPTH)
            pltpu.make_async_copy(
                tt_hbm.at[:, pl.ds(0, 128)],
                bufs[slot],
                sems[slot],
            ).wait()
            rb = z16 + rv[k]
            tb = z16 + (jg * 16 + k)
            for m in range(_D // 16):
                c16 = lanes + m * 16
                vals = plsc.load_gather(bufs[slot], [c16, rb])
                plsc.store_scatter(stg_v, [c16, tb], vals)
        return carry

    lax.fori_loop(0, _NGRP, group, 0)

    for t in range(_DEPTH - 1):
        slot = (_B_PER_W + t) % _DEPTH
        pltpu.make_async_copy(
            tt_hbm.at[:, pl.ds(0, 128)],
            bufs[slot],
            sems[slot],
        ).wait()

    pltpu.sync_copy(stg_v, out_hbm.at[:, pl.ds(base, _B_PER_W)])


@jax.jit
def _gather(tt, idx):
    mesh = plsc.VectorSubcoreMesh(core_axis_name="c", subcore_axis_name="s")
    return pl.kernel(
        _gather_body,
        mesh=mesh,
        out_type=jax.ShapeDtypeStruct((_D, _B), jnp.float32),
        scratch_types=[
            pltpu.VMEM((_B_PER_W,), jnp.int32),
            pltpu.VMEM((_B_PER_W,), jnp.int32),
            pltpu.VMEM((_B_PER_W,), jnp.int32),
            pltpu.VMEM((_D, _B_PER_W), jnp.float32),
            pltpu.VMEM((_D, 128), jnp.float32),
            pltpu.VMEM((_D, 128), jnp.float32),
            pltpu.VMEM((_D, 128), jnp.float32),
            pltpu.VMEM((_D, 128), jnp.float32),
            pltpu.VMEM((_D, 128), jnp.float32),
            pltpu.VMEM((_D, 128), jnp.float32),
            pltpu.VMEM((_D, 128), jnp.float32),
            pltpu.VMEM((_D, 128), jnp.float32),
            pltpu.SemaphoreType.DMA,
            pltpu.SemaphoreType.DMA,
            pltpu.SemaphoreType.DMA,
            pltpu.SemaphoreType.DMA,
            pltpu.SemaphoreType.DMA,
            pltpu.SemaphoreType.DMA,
            pltpu.SemaphoreType.DMA,
            pltpu.SemaphoreType.DMA,
        ],
        compiler_params=pltpu.CompilerParams(
            use_tc_tiling_on_sc=True, needs_layout_passes=False),
    )(tt, idx)


def kernel(tensor, inds):
    out_t = _gather(tensor.T, jnp.squeeze(inds, axis=1))
    return out_t.T
